# baseline (speedup 1.0000x reference)
import jax
import jax.numpy as jnp
from jax import lax
from jax.experimental import pallas as pl
from jax.experimental.pallas import tpu as pltpu
from jax.experimental.pallas import tpu_sc as plsc

NUM_FIELDS = 26
VOCAB = 100000
EMBED_DIM = 32
BATCH = 16384

NC, NS = 2, 16
NW = NC * NS
BBLK = 1024
CBLK = BATCH // BBLK
GROUPS = (2, 8, 8, 8)

_MESH = plsc.VectorSubcoreMesh(core_axis_name="c", subcore_axis_name="s")


def _make_gather(g):
    ch_per_w = g * CBLK // NW

    def body(fT_hbm, tT_hbm, o_hbm, idx_v, o_v, sem):
        wid = lax.axis_index("s") * NC + lax.axis_index("c")

        def chunk(c, carry):
            cid = wid * ch_per_w + c
            f = cid // CBLK
            cb = cid % CBLK
            pltpu.sync_copy(fT_hbm.at[f, pl.ds(cb * BBLK, BBLK)], idx_v)
            copies = [
                pltpu.async_copy(tT_hbm.at[f, e].at[idx_v], o_v.at[e], sem)
                for e in range(EMBED_DIM)
            ]
            for cp in copies:
                cp.wait()
            pltpu.sync_copy(
                o_v, o_hbm.at[pl.ds(32 * f, 32), pl.ds(cb * BBLK, BBLK)])
            return carry

        lax.fori_loop(0, ch_per_w, chunk, 0)

    return pl.kernel(
        body,
        out_type=jax.ShapeDtypeStruct((g * EMBED_DIM, BATCH), jnp.float32),
        mesh=_MESH,
        compiler_params=pltpu.CompilerParams(use_tc_tiling_on_sc=False),
        scratch_types=[
            pltpu.VMEM((BBLK,), jnp.int32),
            pltpu.VMEM((EMBED_DIM, BBLK), jnp.float32),
            pltpu.SemaphoreType.DMA,
        ],
    )


_CALLS = {g: _make_gather(g) for g in set(GROUPS)}


def kernel(features, tables):
    tT = tables.transpose(0, 2, 1)
    fT = features.astype(jnp.int32).T
    outs = []
    off = 0
    for g in GROUPS:
        outs.append(_CALLS[g](fT[off:off + g], tT[off:off + g]))
        off += g
    o = jnp.concatenate(outs, axis=0)
    return o.T


# speedup vs baseline: 1.0491x; 1.0491x over previous
import jax
import jax.numpy as jnp
from jax import lax
from jax.experimental import pallas as pl
from jax.experimental.pallas import tpu as pltpu
from jax.experimental.pallas import tpu_sc as plsc

NUM_FIELDS = 26
VOCAB = 100000
EMBED_DIM = 32
BATCH = 16384

NC, NS = 2, 16
NW = NC * NS
BBLK = 1024
CBLK = BATCH // BBLK
GROUPS = (2,) * 13

_MESH = plsc.VectorSubcoreMesh(core_axis_name="c", subcore_axis_name="s")


def _make_gather(g):
    ch_per_w = g * CBLK // NW

    def body(fT_hbm, tT_hbm, o_hbm, idx_v, o_v, sem):
        wid = lax.axis_index("s") * NC + lax.axis_index("c")

        def chunk(c, carry):
            cid = wid * ch_per_w + c
            f = cid // CBLK
            cb = cid % CBLK
            pltpu.sync_copy(fT_hbm.at[f, pl.ds(cb * BBLK, BBLK)], idx_v)
            copies = [
                pltpu.async_copy(tT_hbm.at[f, e].at[idx_v], o_v.at[e], sem)
                for e in range(EMBED_DIM)
            ]
            for cp in copies:
                cp.wait()
            pltpu.sync_copy(
                o_v, o_hbm.at[pl.ds(32 * f, 32), pl.ds(cb * BBLK, BBLK)])
            return carry

        lax.fori_loop(0, ch_per_w, chunk, 0)

    return pl.kernel(
        body,
        out_type=jax.ShapeDtypeStruct((g * EMBED_DIM, BATCH), jnp.float32),
        mesh=_MESH,
        compiler_params=pltpu.CompilerParams(use_tc_tiling_on_sc=False),
        scratch_types=[
            pltpu.VMEM((BBLK,), jnp.int32),
            pltpu.VMEM((EMBED_DIM, BBLK), jnp.float32),
            pltpu.SemaphoreType.DMA,
        ],
    )


_CALLS = {g: _make_gather(g) for g in set(GROUPS)}


def kernel(features, tables):
    tT = tables.transpose(0, 2, 1)
    fT = features.astype(jnp.int32).T
    outs = []
    off = 0
    for g in GROUPS:
        outs.append(_CALLS[g](fT[off:off + g], tT[off:off + g]))
        off += g
    o = jnp.concatenate(outs, axis=0)
    return o.T
